# baseline (speedup 1.0000x reference)
import functools

import jax
import jax.numpy as jnp
from jax import lax
from jax.experimental import pallas as pl
from jax.experimental.pallas import tpu as pltpu
from jax.experimental.pallas import tpu_sc as plsc

N = 518400
B = 65536
C = 49
KT = N // 128
NKC = B // 128
KTOT = KT + NKC
SP = 7
PLANE = KTOT * 1024
FLAT = SP * PLANE
OPACITY_P = 13

NC = 2
NS = 16
NW = NC * NS
BPW = B // NW
CHUNK = 128
NCHUNK = BPW // CHUNK

_POFF = [(p // 8) * PLANE + (p % 8) * 128 for p in range(C)]
_GLEN = (KT - 1) * 1024 + 128


def _t_body(o_ref, t_ref):
    x = o_ref[...].reshape(KT, 128)
    p = jnp.clip(jax.nn.sigmoid(x) * 0.5, 1e-4, 1.0 - 1e-4)
    t_ref[...] = jnp.log(p) - jnp.log1p(-p)


_t_call = pl.pallas_call(
    _t_body,
    out_shape=jax.ShapeDtypeStruct((KT, 128), jnp.float32),
    in_specs=[pl.BlockSpec((KT, 1, 128), lambda: (0, 0, 0))],
    out_specs=pl.BlockSpec((KT, 128), lambda: (0, 0)),
)


def _idx_body(i_ref, b_ref, s_ref):
    iv = i_ref[...]
    br = ((iv >> 7) << 10) | (iv & 127)
    b_ref[...] = br
    s_ref[...] = br + _POFF[OPACITY_P]


_idx_call = pl.pallas_call(
    _idx_body,
    out_shape=[jax.ShapeDtypeStruct((B // 128, 128), jnp.int32),
               jax.ShapeDtypeStruct((B // 128, 128), jnp.int32)],
    in_specs=[pl.BlockSpec((B // 128, 128), lambda: (0, 0))],
    out_specs=[pl.BlockSpec((B // 128, 128), lambda: (0, 0)),
               pl.BlockSpec((B // 128, 128), lambda: (0, 0))],
)


PACK_KB = 81


def _pack_body(uv2, dep2, xyz2, quat2, scl2, opa2, rgb2, lat3, out3):
    kb = PACK_KB

    def planes(ref, w):
        x = ref[...]
        return [x[:, c, :] for c in range(w)]

    ps = (planes(uv2, 2) + planes(dep2, 1) + planes(xyz2, 3)
          + planes(quat2, 4) + planes(scl2, 3) + planes(opa2, 1)
          + planes(rgb2, 3))
    lat = lat3[...].reshape(4, kb, 8, 128)
    ps += [lat[c // 8, :, c % 8, :] for c in range(32)]
    zero = jnp.zeros((kb, 128), jnp.float32)
    for g in range(SP):
        y = jnp.stack([ps[8 * g + s] if 8 * g + s < C else zero
                       for s in range(8)], axis=1)
        out3[g] = y.reshape(kb * 8, 128)


_pack_call = pl.pallas_call(
    _pack_body,
    grid=((KT + PACK_KB - 1) // PACK_KB,),
    in_specs=[
        pl.BlockSpec((PACK_KB, 2, 128), lambda i: (i, 0, 0)),
        pl.BlockSpec((PACK_KB, 1, 128), lambda i: (i, 0, 0)),
        pl.BlockSpec((PACK_KB, 3, 128), lambda i: (i, 0, 0)),
        pl.BlockSpec((PACK_KB, 4, 128), lambda i: (i, 0, 0)),
        pl.BlockSpec((PACK_KB, 3, 128), lambda i: (i, 0, 0)),
        pl.BlockSpec((PACK_KB, 1, 128), lambda i: (i, 0, 0)),
        pl.BlockSpec((PACK_KB, 3, 128), lambda i: (i, 0, 0)),
        pl.BlockSpec((4, PACK_KB * 8, 128), lambda i: (0, i, 0)),
    ],
    out_specs=pl.BlockSpec((SP, PACK_KB * 8, 128), lambda i: (0, i, 0)),
    out_shape=jax.ShapeDtypeStruct((SP, KTOT * 8, 128), jnp.float32),
    compiler_params=pltpu.CompilerParams(dimension_semantics=("parallel",)),
)


def _make_clone_kernel(j0, j1):
  nj = j1 - j0

  @functools.partial(
      pl.kernel,
      out_type=(),
      mesh=plsc.VectorSubcoreMesh(core_axis_name="c", subcore_axis_name="s"),
      compiler_params=pltpu.CompilerParams(needs_layout_passes=False,
                                           use_tc_tiling_on_sc=False),
      scratch_types=[
          pltpu.VMEM((nj, CHUNK), jnp.int32),
          pltpu.VMEM((nj, CHUNK), jnp.int32),
          pltpu.VMEM((nj, CHUNK), jnp.int32),
          pltpu.VMEM((3, C, CHUNK), jnp.float32),
          pltpu.VMEM((3, CHUNK), jnp.float32),
          pltpu.SemaphoreType.DMA,
          pltpu.SemaphoreType.DMA,
      ],
  )
  def _clone_kernel(flat_ref, t_hbm, idx_hbm, bidx_hbm, sidx_hbm, idx_v,
                    bidx_v, sidx_v, dst2_v, tv2_v, gsem, wsem):
    wid = lax.axis_index("s") * NC + lax.axis_index("c")
    pltpu.sync_copy(idx_hbm.at[wid, pl.ds(j0, nj)], idx_v)
    pltpu.sync_copy(bidx_hbm.at[wid, pl.ds(j0, nj)], bidx_v)
    pltpu.sync_copy(sidx_hbm.at[wid, pl.ds(j0, nj)], sidx_v)
    gplanes = [p for p in range(C) if p != OPACITY_P]

    def issue_gathers(jj):
        dst_v = dst2_v.at[jj % 3]
        tv_v = tv2_v.at[jj % 3]
        gs = [pltpu.async_copy(t_hbm.at[idx_v.at[jj]], tv_v, gsem)]
        for p in gplanes:
            src = flat_ref.at[pl.ds(_POFF[p], _GLEN)].at[bidx_v.at[jj]]
            gs.append(pltpu.async_copy(src, dst_v.at[p], gsem))
        return gs

    def issue_writes(jj):
        dst_v = dst2_v.at[jj % 3]
        tv_v = tv2_v.at[jj % 3]
        kbase = (KT + wid * NCHUNK + j0 + jj) * 1024
        ws = []
        for p in range(C):
            src = tv_v if p == OPACITY_P else dst_v.at[p]
            dst = flat_ref.at[pl.ds(pl.multiple_of(_POFF[p] + kbase, 128),
                                    CHUNK)]
            ws.append(pltpu.async_copy(src, dst, wsem))
        ws.append(pltpu.async_copy(tv_v, flat_ref.at[sidx_v.at[jj]], wsem))
        return ws

    pending_g = {0: issue_gathers(0)}
    pending_w = {}
    for jj in range(nj):
        if jj + 1 < nj:
            if jj - 2 in pending_w:
                for w in pending_w.pop(jj - 2):
                    w.wait()
            pending_g[jj + 1] = issue_gathers(jj + 1)
        for g in pending_g.pop(jj):
            g.wait()
        pending_w[jj] = issue_writes(jj)
    for k in sorted(pending_w):
        for w in pending_w[k]:
            w.wait()

  return _clone_kernel


_clone_a = _make_clone_kernel(0, NCHUNK // 2)
_clone_b = _make_clone_kernel(NCHUNK // 2, NCHUNK)


def kernel(uv, depth_raw, xyz_offset, quat_raw, log_scale, opacity_logit,
           rgb_logit, latent, indices):
    uv2 = uv.reshape(KT, 128, 2).transpose(0, 2, 1)
    dep2 = depth_raw.reshape(KT, 128, 1).transpose(0, 2, 1)
    xyz2 = xyz_offset.reshape(KT, 128, 3).transpose(0, 2, 1)
    quat2 = quat_raw.reshape(KT, 128, 4).transpose(0, 2, 1)
    scl2 = log_scale.reshape(KT, 128, 3).transpose(0, 2, 1)
    opa2 = opacity_logit.reshape(KT, 128, 1).transpose(0, 2, 1)
    rgb2 = rgb_logit.reshape(KT, 128, 3).transpose(0, 2, 1)
    lat3 = latent.reshape(KT, 128, 4, 8).transpose(2, 0, 3, 1).reshape(4, KT * 8, 128)

    t = _t_call(opa2).reshape(N)
    p4 = _pack_call(uv2, dep2, xyz2, quat2, scl2, opa2, rgb2, lat3)

    bidx, sidx = _idx_call(indices.reshape(B // 128, 128))
    idx3 = indices.reshape(NW, NCHUNK, CHUNK)
    bidx3 = bidx.reshape(NW, NCHUNK, CHUNK)
    sidx3 = sidx.reshape(NW, NCHUNK, CHUNK)
    ref = jax.new_ref(p4.reshape(FLAT))
    _clone_a(ref, t, idx3, bidx3, sidx3)
    _clone_b(ref, t, idx3, bidx3, sidx3)
    out = jax.freeze(ref).reshape(SP, KTOT, 8, 128)
    return out.transpose(1, 3, 0, 2).reshape(KTOT * 128, SP * 8)[:, :C]


# speedup vs baseline: 1.0166x; 1.0166x over previous
import functools

import jax
import jax.numpy as jnp
from jax import lax
from jax.experimental import pallas as pl
from jax.experimental.pallas import tpu as pltpu
from jax.experimental.pallas import tpu_sc as plsc

N = 518400
B = 65536
C = 49
KT = N // 128
NKC = B // 128
KTOT = KT + NKC
SP = 7
PLANE = KTOT * 1024
FLAT = SP * PLANE
OPACITY_P = 13

NC = 2
NS = 16
NW = NC * NS
BPW = B // NW
CHUNK = 128
NCHUNK = BPW // CHUNK

_POFF = [(p // 8) * PLANE + (p % 8) * 128 for p in range(C)]
_GLEN = (KT - 1) * 1024 + 128


def _t_body(o_ref, t_ref):
    x = o_ref[...].reshape(KT, 128)
    p = jnp.clip(jax.nn.sigmoid(x) * 0.5, 1e-4, 1.0 - 1e-4)
    t_ref[...] = jnp.log(p) - jnp.log1p(-p)


_t_call = pl.pallas_call(
    _t_body,
    out_shape=jax.ShapeDtypeStruct((KT, 128), jnp.float32),
    in_specs=[pl.BlockSpec((KT, 1, 128), lambda: (0, 0, 0))],
    out_specs=pl.BlockSpec((KT, 128), lambda: (0, 0)),
)


def _idx_body(i_ref, b_ref, s_ref):
    iv = i_ref[...]
    br = ((iv >> 7) << 10) | (iv & 127)
    b_ref[...] = br
    s_ref[...] = br + _POFF[OPACITY_P]


_idx_call = pl.pallas_call(
    _idx_body,
    out_shape=[jax.ShapeDtypeStruct((B // 128, 128), jnp.int32),
               jax.ShapeDtypeStruct((B // 128, 128), jnp.int32)],
    in_specs=[pl.BlockSpec((B // 128, 128), lambda: (0, 0))],
    out_specs=[pl.BlockSpec((B // 128, 128), lambda: (0, 0)),
               pl.BlockSpec((B // 128, 128), lambda: (0, 0))],
)


PACK_KB = 81


def _pack_body(uv2, dep2, xyz2, quat2, scl2, opa2, rgb2, lat3, out3):
    kb = PACK_KB

    def planes(ref, w):
        x = ref[...]
        return [x[:, c, :] for c in range(w)]

    ps = (planes(uv2, 2) + planes(dep2, 1) + planes(xyz2, 3)
          + planes(quat2, 4) + planes(scl2, 3) + planes(opa2, 1)
          + planes(rgb2, 3))
    lat = lat3[...].reshape(4, kb, 8, 128)
    ps += [lat[c // 8, :, c % 8, :] for c in range(32)]
    zero = jnp.zeros((kb, 128), jnp.float32)
    for g in range(SP):
        y = jnp.stack([ps[8 * g + s] if 8 * g + s < C else zero
                       for s in range(8)], axis=1)
        out3[g] = y.reshape(kb * 8, 128)


_pack_call = pl.pallas_call(
    _pack_body,
    grid=((KT + PACK_KB - 1) // PACK_KB,),
    in_specs=[
        pl.BlockSpec((PACK_KB, 2, 128), lambda i: (i, 0, 0)),
        pl.BlockSpec((PACK_KB, 1, 128), lambda i: (i, 0, 0)),
        pl.BlockSpec((PACK_KB, 3, 128), lambda i: (i, 0, 0)),
        pl.BlockSpec((PACK_KB, 4, 128), lambda i: (i, 0, 0)),
        pl.BlockSpec((PACK_KB, 3, 128), lambda i: (i, 0, 0)),
        pl.BlockSpec((PACK_KB, 1, 128), lambda i: (i, 0, 0)),
        pl.BlockSpec((PACK_KB, 3, 128), lambda i: (i, 0, 0)),
        pl.BlockSpec((4, PACK_KB * 8, 128), lambda i: (0, i, 0)),
    ],
    out_specs=pl.BlockSpec((SP, PACK_KB * 8, 128), lambda i: (0, i, 0)),
    out_shape=jax.ShapeDtypeStruct((SP, KTOT * 8, 128), jnp.float32),
    compiler_params=pltpu.CompilerParams(dimension_semantics=("parallel",)),
)


def _make_clone_kernel(j0, j1):
  nj = j1 - j0

  @functools.partial(
      pl.kernel,
      out_type=(),
      mesh=plsc.VectorSubcoreMesh(core_axis_name="c", subcore_axis_name="s"),
      compiler_params=pltpu.CompilerParams(needs_layout_passes=False,
                                           use_tc_tiling_on_sc=False),
      scratch_types=[
          pltpu.VMEM((nj, CHUNK), jnp.int32),
          pltpu.VMEM((nj, CHUNK), jnp.int32),
          pltpu.VMEM((nj, CHUNK), jnp.int32),
          pltpu.VMEM((2, C, CHUNK), jnp.float32),
          pltpu.VMEM((2, CHUNK), jnp.float32),
          pltpu.SemaphoreType.DMA,
          pltpu.SemaphoreType.DMA,
      ],
  )
  def _clone_kernel(flat_ref, t_hbm, idx_hbm, bidx_hbm, sidx_hbm, idx_v,
                    bidx_v, sidx_v, dst2_v, tv2_v, gsem, wsem):
    wid = lax.axis_index("s") * NC + lax.axis_index("c")
    pltpu.sync_copy(idx_hbm.at[wid, pl.ds(j0, nj)], idx_v)
    pltpu.sync_copy(bidx_hbm.at[wid, pl.ds(j0, nj)], bidx_v)
    pltpu.sync_copy(sidx_hbm.at[wid, pl.ds(j0, nj)], sidx_v)
    gplanes = [p for p in range(C) if p != OPACITY_P]
    prev_writes = []
    for jj in range(nj):
        dst_v = dst2_v.at[jj % 2]
        tv_v = tv2_v.at[jj % 2]
        gathers = [pltpu.async_copy(t_hbm.at[idx_v.at[jj]], tv_v, gsem)]
        for p in gplanes:
            src = flat_ref.at[pl.ds(_POFF[p], _GLEN)].at[bidx_v.at[jj]]
            gathers.append(pltpu.async_copy(src, dst_v.at[p], gsem))
        for w in prev_writes:
            w.wait()
        for g in gathers:
            g.wait()
        kbase = (KT + wid * NCHUNK + j0 + jj) * 1024
        writes = []
        for p in range(C):
            src = tv_v if p == OPACITY_P else dst_v.at[p]
            dst = flat_ref.at[pl.ds(pl.multiple_of(_POFF[p] + kbase, 128),
                                    CHUNK)]
            writes.append(pltpu.async_copy(src, dst, wsem))
        writes.append(pltpu.async_copy(tv_v, flat_ref.at[sidx_v.at[jj]], wsem))
        prev_writes = writes
    for w in prev_writes:
        w.wait()

  return _clone_kernel


_clone_a = _make_clone_kernel(0, NCHUNK // 2)
_clone_b = _make_clone_kernel(NCHUNK // 2, NCHUNK)


def kernel(uv, depth_raw, xyz_offset, quat_raw, log_scale, opacity_logit,
           rgb_logit, latent, indices):
    uv2 = uv.reshape(KT, 128, 2).transpose(0, 2, 1)
    dep2 = depth_raw.reshape(KT, 128, 1).transpose(0, 2, 1)
    xyz2 = xyz_offset.reshape(KT, 128, 3).transpose(0, 2, 1)
    quat2 = quat_raw.reshape(KT, 128, 4).transpose(0, 2, 1)
    scl2 = log_scale.reshape(KT, 128, 3).transpose(0, 2, 1)
    opa2 = opacity_logit.reshape(KT, 128, 1).transpose(0, 2, 1)
    rgb2 = rgb_logit.reshape(KT, 128, 3).transpose(0, 2, 1)
    lat3 = latent.reshape(KT, 128, 4, 8).transpose(2, 0, 3, 1).reshape(4, KT * 8, 128)

    t = _t_call(opa2).reshape(N)
    p4 = _pack_call(uv2, dep2, xyz2, quat2, scl2, opa2, rgb2, lat3)

    bidx, sidx = _idx_call(indices.reshape(B // 128, 128))
    idx3 = indices.reshape(NW, NCHUNK, CHUNK)
    bidx3 = bidx.reshape(NW, NCHUNK, CHUNK)
    sidx3 = sidx.reshape(NW, NCHUNK, CHUNK)
    ref = jax.new_ref(p4.reshape(FLAT))
    _clone_a(ref, t, idx3, bidx3, sidx3)
    _clone_b(ref, t, idx3, bidx3, sidx3)
    out = jax.freeze(ref).reshape(SP, KTOT, 8, 128)
    return out.transpose(1, 3, 0, 2).reshape(KTOT * 128, SP * 8)[:, :C]
